# baseline (speedup 1.0000x reference)
import functools

import jax
import jax.numpy as jnp
import numpy as np
from jax import lax
from jax.experimental import pallas as pl
from jax.experimental.pallas import tpu as pltpu
from jax.experimental.pallas import tpu_sc as plsc

N = 2048
D = 768
F = 2048
E = 8
TILE = 128
NPAD = N + E * TILE
NT = NPAD // TILE
NW = 32
TPW = N // NW

_GELU_C = np.float32(np.sqrt(2.0 / np.pi))


def _router_body(lg_ref, dest_ref, te_ref, aux_ref, oh_ref):
    logits = lg_ref[...]
    m = jnp.max(logits, axis=1, keepdims=True)
    ex = jnp.exp(logits - m)
    probs = ex / jnp.sum(ex, axis=1, keepdims=True)

    maxp = jnp.max(probs, axis=1, keepdims=True)
    ism = (probs == maxp).astype(jnp.float32)
    r8 = lax.broadcasted_iota(jnp.int32, (E, E), 0)
    c8 = lax.broadcasted_iota(jnp.int32, (E, E), 1)
    upper = (r8 <= c8).astype(jnp.float32)
    pref = jnp.dot(ism, upper, preferred_element_type=jnp.float32)
    onehot = ism * (pref == 1.0).astype(jnp.float32)
    oh_ref[...] = onehot

    counts = jnp.sum(onehot, axis=0, keepdims=True)
    pbar = jnp.mean(probs, axis=0, keepdims=True)
    aux_ref[...] = jnp.float32(E) * jnp.sum(
        counts / jnp.float32(N) * pbar, keepdims=True
    )

    rt = lax.broadcasted_iota(jnp.int32, (TILE, TILE), 0)
    ct = lax.broadcasted_iota(jnp.int32, (TILE, TILE), 1)
    tri = (rt >= ct).astype(jnp.float32)

    def body(i, base):
        chunk = oh_ref[pl.ds(i * TILE, TILE), :]
        cs = jnp.dot(tri, chunk, preferred_element_type=jnp.float32) + base
        oh_ref[pl.ds(i * TILE, TILE), :] = cs
        return cs[TILE - 1:TILE, :]

    lax.fori_loop(0, N // TILE, body, jnp.zeros((1, E), jnp.float32))
    cum = oh_ref[...]

    pc = jnp.ceil(counts / jnp.float32(TILE)) * jnp.float32(TILE)
    ends = jnp.dot(pc, upper, preferred_element_type=jnp.float32)
    po = ends - pc
    dest_f = jnp.sum(onehot * (cum + po - 1.0), axis=1, keepdims=True)
    dest_ref[...] = dest_f.astype(jnp.int32)

    tcut = lax.broadcasted_iota(jnp.int32, (NT, 1), 0).astype(jnp.float32) * jnp.float32(TILE)
    done = (jnp.broadcast_to(ends, (NT, E)) <= tcut).astype(jnp.float32)
    te = jnp.clip(jnp.sum(done, axis=1, keepdims=True), 0.0, jnp.float32(E - 1))
    te_ref[...] = te.astype(jnp.int32)


def _router(logits):
    return pl.pallas_call(
        _router_body,
        out_shape=[
            jax.ShapeDtypeStruct((N, 1), jnp.int32),
            jax.ShapeDtypeStruct((NT, 1), jnp.int32),
            jax.ShapeDtypeStruct((1, 1), jnp.float32),
        ],
        scratch_shapes=[pltpu.VMEM((N, E), jnp.float32)],
    )(logits)


@functools.cache
def _sc_kernels():
    mesh = plsc.VectorSubcoreMesh(core_axis_name="c", subcore_axis_name="s")

    @functools.partial(
        pl.kernel,
        out_type=jax.ShapeDtypeStruct((NPAD, D), jnp.float32),
        mesh=mesh,
        scratch_types=[
            pltpu.VMEM((TPW,), jnp.int32),
            pltpu.VMEM((TPW, D), jnp.float32),
            pltpu.SemaphoreType.DMA,
        ],
    )
    def sc_scatter(x_hbm, dest_hbm, xs_hbm, idx_v, rows_v, sem):
        wid = lax.axis_index("s") * 2 + lax.axis_index("c")
        base = wid * TPW
        pltpu.sync_copy(dest_hbm.at[pl.ds(base, TPW)], idx_v)
        pltpu.sync_copy(x_hbm.at[pl.ds(base, TPW)], rows_v)
        pltpu.async_copy(rows_v, xs_hbm.at[idx_v], sem).wait()

    @functools.partial(
        pl.kernel,
        out_type=jax.ShapeDtypeStruct((N, D), jnp.float32),
        mesh=mesh,
        scratch_types=[
            pltpu.VMEM((TPW,), jnp.int32),
            pltpu.VMEM((TPW, D), jnp.float32),
            pltpu.SemaphoreType.DMA,
        ],
    )
    def sc_gather(ys_hbm, dest_hbm, out_hbm, idx_v, rows_v, sem):
        wid = lax.axis_index("s") * 2 + lax.axis_index("c")
        base = wid * TPW
        pltpu.sync_copy(dest_hbm.at[pl.ds(base, TPW)], idx_v)
        pltpu.async_copy(ys_hbm.at[idx_v], rows_v, sem).wait()
        pltpu.sync_copy(rows_v, out_hbm.at[pl.ds(base, TPW)])

    return sc_scatter, sc_gather


def _ffn_body(te_ref, xs_ref, w1_ref, w2_ref, ys_ref, w1q_ref, w2q_ref):
    i = pl.program_id(0)
    cur = te_ref[i]
    prev = te_ref[jnp.maximum(i - 1, 0)]

    @pl.when((i == 0) | (cur != prev))
    def _quantize_weights():
        inv_n = jnp.float32(1.0 / (D * F))
        w1 = w1_ref[0]
        s1 = 1.0 / jnp.clip(jnp.sum(jnp.abs(w1)) * inv_n, 1e-5)
        w1q_ref[...] = (jnp.clip(jnp.round(w1 * s1), -1.0, 1.0) / s1).astype(jnp.bfloat16)
        w2 = w2_ref[0]
        s2 = 1.0 / jnp.clip(jnp.sum(jnp.abs(w2)) * inv_n, 1e-5)
        w2q_ref[...] = (jnp.clip(jnp.round(w2 * s2), -1.0, 1.0) / s2).astype(jnp.bfloat16)

    x = xs_ref[...]
    sx = 127.0 / jnp.clip(jnp.max(jnp.abs(x), axis=1, keepdims=True), 1e-5)
    xq = (jnp.clip(jnp.round(x * sx), -128.0, 127.0) / sx).astype(jnp.bfloat16)
    h = jnp.dot(xq, w1q_ref[...], preferred_element_type=jnp.float32)
    g = 0.5 * h * (1.0 + jnp.tanh(_GELU_C * (h + 0.044715 * h * h * h)))
    sg = 127.0 / jnp.clip(jnp.max(jnp.abs(g), axis=1, keepdims=True), 1e-5)
    gq = (jnp.clip(jnp.round(g * sg), -128.0, 127.0) / sg).astype(jnp.bfloat16)
    y = jnp.dot(gq, w2q_ref[...], preferred_element_type=jnp.float32)
    ys_ref[...] = y


def _ffn(te, xs, w1, w2):
    grid_spec = pltpu.PrefetchScalarGridSpec(
        num_scalar_prefetch=1,
        grid=(NT,),
        in_specs=[
            pl.BlockSpec((TILE, D), lambda i, te: (i, 0)),
            pl.BlockSpec((1, D, F), lambda i, te: (te[i], 0, 0)),
            pl.BlockSpec((1, F, D), lambda i, te: (te[i], 0, 0)),
        ],
        out_specs=pl.BlockSpec((TILE, D), lambda i, te: (i, 0)),
        scratch_shapes=[
            pltpu.VMEM((D, F), jnp.bfloat16),
            pltpu.VMEM((F, D), jnp.bfloat16),
        ],
    )
    return pl.pallas_call(
        _ffn_body,
        grid_spec=grid_spec,
        out_shape=jax.ShapeDtypeStruct((NPAD, D), jnp.float32),
        compiler_params=pltpu.CompilerParams(
            dimension_semantics=("arbitrary",),
            vmem_limit_bytes=100 * 1024 * 1024,
        ),
    )(te, xs, w1, w2)


def kernel(x, router_w, w1, w2):
    sc_scatter, sc_gather = _sc_kernels()
    xf = x.reshape(N, D)
    logits = jnp.dot(xf, router_w)
    dest2d, te2d, aux = _router(logits)
    dest = dest2d.reshape(N)
    te = te2d.reshape(NT)
    xs = sc_scatter(xf, dest)
    ys = _ffn(te, xs, w1, w2)
    out = sc_gather(ys, dest)
    return out.reshape(x.shape), aux.reshape(())


# speedup vs baseline: 1.0216x; 1.0216x over previous
import functools

import jax
import jax.numpy as jnp
import numpy as np
from jax import lax
from jax.experimental import pallas as pl
from jax.experimental.pallas import tpu as pltpu
from jax.experimental.pallas import tpu_sc as plsc

N = 2048
D = 768
F = 2048
E = 8
TILE = 128
NPAD = N + E * TILE
NT = NPAD // TILE
NW = 32
TPW = N // NW

_GELU_C = np.float32(np.sqrt(2.0 / np.pi))


def _router_body(lg_ref, dest_ref, te_ref, aux_ref, oh_ref):
    logits = lg_ref[...]
    m = jnp.max(logits, axis=1, keepdims=True)
    ex = jnp.exp(logits - m)
    probs = ex / jnp.sum(ex, axis=1, keepdims=True)

    maxp = jnp.max(probs, axis=1, keepdims=True)
    ism = (probs == maxp).astype(jnp.float32)
    r8 = lax.broadcasted_iota(jnp.int32, (E, E), 0)
    c8 = lax.broadcasted_iota(jnp.int32, (E, E), 1)
    upper = (r8 <= c8).astype(jnp.float32)
    pref = jnp.dot(ism, upper, preferred_element_type=jnp.float32)
    onehot = ism * (pref == 1.0).astype(jnp.float32)
    oh_ref[...] = onehot

    counts = jnp.sum(onehot, axis=0, keepdims=True)
    pbar = jnp.mean(probs, axis=0, keepdims=True)
    aux_ref[...] = jnp.float32(E) * jnp.sum(
        counts / jnp.float32(N) * pbar, keepdims=True
    )

    rt = lax.broadcasted_iota(jnp.int32, (TILE, TILE), 0)
    ct = lax.broadcasted_iota(jnp.int32, (TILE, TILE), 1)
    tri = (rt >= ct).astype(jnp.float32)

    def body(i, base):
        chunk = oh_ref[pl.ds(i * TILE, TILE), :]
        cs = jnp.dot(tri, chunk, preferred_element_type=jnp.float32) + base
        oh_ref[pl.ds(i * TILE, TILE), :] = cs
        return cs[TILE - 1:TILE, :]

    lax.fori_loop(0, N // TILE, body, jnp.zeros((1, E), jnp.float32))
    cum = oh_ref[...]

    pc = jnp.ceil(counts / jnp.float32(TILE)) * jnp.float32(TILE)
    ends = jnp.dot(pc, upper, preferred_element_type=jnp.float32)
    po = ends - pc
    dest_f = jnp.sum(onehot * (cum + po - 1.0), axis=1, keepdims=True)
    dest_ref[...] = dest_f.astype(jnp.int32)

    tcut = lax.broadcasted_iota(jnp.int32, (NT, 1), 0).astype(jnp.float32) * jnp.float32(TILE)
    done = (jnp.broadcast_to(ends, (NT, E)) <= tcut).astype(jnp.float32)
    te = jnp.clip(jnp.sum(done, axis=1, keepdims=True), 0.0, jnp.float32(E - 1))
    n_used = ends[:, E - 1:E] / jnp.float32(TILE)
    te_ref[...] = jnp.concatenate([te, n_used], axis=0).astype(jnp.int32)


def _router(logits):
    return pl.pallas_call(
        _router_body,
        out_shape=[
            jax.ShapeDtypeStruct((N, 1), jnp.int32),
            jax.ShapeDtypeStruct((NT + 1, 1), jnp.int32),
            jax.ShapeDtypeStruct((1, 1), jnp.float32),
        ],
        scratch_shapes=[pltpu.VMEM((N, E), jnp.float32)],
    )(logits)


@functools.cache
def _sc_kernels():
    mesh = plsc.VectorSubcoreMesh(core_axis_name="c", subcore_axis_name="s")

    @functools.partial(
        pl.kernel,
        out_type=jax.ShapeDtypeStruct((NPAD, D), jnp.float32),
        mesh=mesh,
        scratch_types=[
            pltpu.VMEM((TPW,), jnp.int32),
            pltpu.VMEM((TPW, D), jnp.float32),
            pltpu.SemaphoreType.DMA,
        ],
    )
    def sc_scatter(x_hbm, dest_hbm, xs_hbm, idx_v, rows_v, sem):
        wid = lax.axis_index("s") * 2 + lax.axis_index("c")
        base = wid * TPW
        pltpu.sync_copy(dest_hbm.at[pl.ds(base, TPW)], idx_v)
        pltpu.sync_copy(x_hbm.at[pl.ds(base, TPW)], rows_v)
        pltpu.async_copy(rows_v, xs_hbm.at[idx_v], sem).wait()

    @functools.partial(
        pl.kernel,
        out_type=jax.ShapeDtypeStruct((N, D), jnp.float32),
        mesh=mesh,
        scratch_types=[
            pltpu.VMEM((TPW,), jnp.int32),
            pltpu.VMEM((TPW, D), jnp.float32),
            pltpu.SemaphoreType.DMA,
        ],
    )
    def sc_gather(ys_hbm, dest_hbm, out_hbm, idx_v, rows_v, sem):
        wid = lax.axis_index("s") * 2 + lax.axis_index("c")
        base = wid * TPW
        pltpu.sync_copy(dest_hbm.at[pl.ds(base, TPW)], idx_v)
        pltpu.async_copy(ys_hbm.at[idx_v], rows_v, sem).wait()
        pltpu.sync_copy(rows_v, out_hbm.at[pl.ds(base, TPW)])

    return sc_scatter, sc_gather


def _ffn_body(te_ref, xs_ref, w1_ref, w2_ref, ys_ref, w1q_ref, w2q_ref):
    i = pl.program_id(0)

    @pl.when(i < te_ref[NT])
    def _compute_tile():
        cur = te_ref[i]
        prev = te_ref[jnp.maximum(i - 1, 0)]

        @pl.when((i == 0) | (cur != prev))
        def _quantize_weights():
            inv_n = jnp.float32(1.0 / (D * F))
            w1 = w1_ref[0]
            s1 = 1.0 / jnp.clip(jnp.sum(jnp.abs(w1)) * inv_n, 1e-5)
            w1q_ref[...] = (jnp.clip(jnp.round(w1 * s1), -1.0, 1.0) / s1).astype(jnp.bfloat16)
            w2 = w2_ref[0]
            s2 = 1.0 / jnp.clip(jnp.sum(jnp.abs(w2)) * inv_n, 1e-5)
            w2q_ref[...] = (jnp.clip(jnp.round(w2 * s2), -1.0, 1.0) / s2).astype(jnp.bfloat16)

        x = xs_ref[...]
        sx = 127.0 / jnp.clip(jnp.max(jnp.abs(x), axis=1, keepdims=True), 1e-5)
        xq = (jnp.clip(jnp.round(x * sx), -128.0, 127.0) / sx).astype(jnp.bfloat16)
        h = jnp.dot(xq, w1q_ref[...], preferred_element_type=jnp.float32)
        g = 0.5 * h * (1.0 + jnp.tanh(_GELU_C * (h + 0.044715 * h * h * h)))
        sg = 127.0 / jnp.clip(jnp.max(jnp.abs(g), axis=1, keepdims=True), 1e-5)
        gq = (jnp.clip(jnp.round(g * sg), -128.0, 127.0) / sg).astype(jnp.bfloat16)
        y = jnp.dot(gq, w2q_ref[...], preferred_element_type=jnp.float32)
        ys_ref[...] = y


def _ffn(te, xs, w1, w2):
    grid_spec = pltpu.PrefetchScalarGridSpec(
        num_scalar_prefetch=1,
        grid=(NT,),
        in_specs=[
            pl.BlockSpec((TILE, D), lambda i, te: (i, 0)),
            pl.BlockSpec((1, D, F), lambda i, te: (te[i], 0, 0)),
            pl.BlockSpec((1, F, D), lambda i, te: (te[i], 0, 0)),
        ],
        out_specs=pl.BlockSpec((TILE, D), lambda i, te: (i, 0)),
        scratch_shapes=[
            pltpu.VMEM((D, F), jnp.bfloat16),
            pltpu.VMEM((F, D), jnp.bfloat16),
        ],
    )
    return pl.pallas_call(
        _ffn_body,
        grid_spec=grid_spec,
        out_shape=jax.ShapeDtypeStruct((NPAD, D), jnp.float32),
        compiler_params=pltpu.CompilerParams(
            dimension_semantics=("arbitrary",),
            vmem_limit_bytes=100 * 1024 * 1024,
        ),
    )(te, xs, w1, w2)


def kernel(x, router_w, w1, w2):
    sc_scatter, sc_gather = _sc_kernels()
    xf = x.reshape(N, D)
    logits = jnp.dot(xf, router_w)
    dest2d, te2d, aux = _router(logits)
    dest = dest2d.reshape(N)
    te = te2d.reshape(NT + 1)
    xs = sc_scatter(xf, dest)
    ys = _ffn(te, xs, w1, w2)
    out = sc_gather(ys, dest)
    return out.reshape(x.shape), aux.reshape(())


# speedup vs baseline: 1.0566x; 1.0343x over previous
import functools

import jax
import jax.numpy as jnp
import numpy as np
from jax import lax
from jax.experimental import pallas as pl
from jax.experimental.pallas import tpu as pltpu
from jax.experimental.pallas import tpu_sc as plsc

N = 2048
D = 768
F = 2048
E = 8
TILE = 128
NPAD = N + E * TILE
NT = NPAD // TILE
NW = 32
TPW = N // NW

_GELU_C = np.float32(np.sqrt(2.0 / np.pi))


def _router_body(lg_ref, dest_ref, te_ref, aux_ref, oh_ref):
    logits = lg_ref[...]
    m = jnp.max(logits, axis=1, keepdims=True)
    ex = jnp.exp(logits - m)
    probs = ex / jnp.sum(ex, axis=1, keepdims=True)

    maxp = jnp.max(probs, axis=1, keepdims=True)
    ism = (probs == maxp).astype(jnp.float32)
    r8 = lax.broadcasted_iota(jnp.int32, (E, E), 0)
    c8 = lax.broadcasted_iota(jnp.int32, (E, E), 1)
    upper = (r8 <= c8).astype(jnp.float32)
    pref = jnp.dot(ism, upper, preferred_element_type=jnp.float32)
    onehot = ism * (pref == 1.0).astype(jnp.float32)
    oh_ref[...] = onehot

    counts = jnp.sum(onehot, axis=0, keepdims=True)
    pbar = jnp.mean(probs, axis=0, keepdims=True)
    aux_ref[...] = jnp.float32(E) * jnp.sum(
        counts / jnp.float32(N) * pbar, keepdims=True
    )

    rt = lax.broadcasted_iota(jnp.int32, (TILE, TILE), 0)
    ct = lax.broadcasted_iota(jnp.int32, (TILE, TILE), 1)
    tri = (rt >= ct).astype(jnp.float32)

    def body(i, base):
        chunk = oh_ref[pl.ds(i * TILE, TILE), :]
        cs = jnp.dot(tri, chunk, preferred_element_type=jnp.float32) + base
        oh_ref[pl.ds(i * TILE, TILE), :] = cs
        return cs[TILE - 1:TILE, :]

    lax.fori_loop(0, N // TILE, body, jnp.zeros((1, E), jnp.float32))
    cum = oh_ref[...]

    pc = jnp.ceil(counts / jnp.float32(TILE)) * jnp.float32(TILE)
    ends = jnp.dot(pc, upper, preferred_element_type=jnp.float32)
    po = ends - pc
    dest_f = jnp.sum(onehot * (cum + po - 1.0), axis=1, keepdims=True)
    dest_ref[...] = dest_f.astype(jnp.int32)

    tcut = lax.broadcasted_iota(jnp.int32, (NT, 1), 0).astype(jnp.float32) * jnp.float32(TILE)
    done = (jnp.broadcast_to(ends, (NT, E)) <= tcut).astype(jnp.float32)
    te = jnp.clip(jnp.sum(done, axis=1, keepdims=True), 0.0, jnp.float32(E - 1))
    n_used = ends[:, E - 1:E] / jnp.float32(TILE)
    te_ref[...] = jnp.concatenate([te, n_used], axis=0).astype(jnp.int32)


def _router(logits):
    return pl.pallas_call(
        _router_body,
        out_shape=[
            jax.ShapeDtypeStruct((N, 1), jnp.int32),
            jax.ShapeDtypeStruct((NT + 1, 1), jnp.int32),
            jax.ShapeDtypeStruct((1, 1), jnp.float32),
        ],
        scratch_shapes=[pltpu.VMEM((N, E), jnp.float32)],
    )(logits)


@functools.cache
def _sc_kernels():
    mesh = plsc.VectorSubcoreMesh(core_axis_name="c", subcore_axis_name="s")

    @functools.partial(
        pl.kernel,
        out_type=jax.ShapeDtypeStruct((NPAD, D), jnp.float32),
        mesh=mesh,
        scratch_types=[
            pltpu.VMEM((TPW,), jnp.int32),
            pltpu.VMEM((TPW, D), jnp.float32),
            pltpu.SemaphoreType.DMA,
        ],
    )
    def sc_scatter(x_hbm, dest_hbm, xs_hbm, idx_v, rows_v, sem):
        wid = lax.axis_index("s") * 2 + lax.axis_index("c")
        base = wid * TPW
        pltpu.sync_copy(dest_hbm.at[pl.ds(base, TPW)], idx_v)
        pltpu.sync_copy(x_hbm.at[pl.ds(base, TPW)], rows_v)
        pltpu.async_copy(rows_v, xs_hbm.at[idx_v], sem).wait()

    @functools.partial(
        pl.kernel,
        out_type=jax.ShapeDtypeStruct((N, D), jnp.float32),
        mesh=mesh,
        scratch_types=[
            pltpu.VMEM((TPW,), jnp.int32),
            pltpu.VMEM((TPW, D), jnp.float32),
            pltpu.SemaphoreType.DMA,
        ],
    )
    def sc_gather(ys_hbm, dest_hbm, out_hbm, idx_v, rows_v, sem):
        wid = lax.axis_index("s") * 2 + lax.axis_index("c")
        base = wid * TPW
        pltpu.sync_copy(dest_hbm.at[pl.ds(base, TPW)], idx_v)
        pltpu.async_copy(ys_hbm.at[idx_v], rows_v, sem).wait()
        pltpu.sync_copy(rows_v, out_hbm.at[pl.ds(base, TPW)])

    return sc_scatter, sc_gather


def _ffn_body(te_ref, xs_ref, w1_ref, w2_ref, ys_ref, w1q_ref, w2q_ref):
    i = pl.program_id(0)

    @pl.when(i < te_ref[NT])
    def _compute_tile():
        cur = te_ref[i]
        prev = te_ref[jnp.maximum(i - 1, 0)]

        @pl.when((i == 0) | (cur != prev))
        def _quantize_weights():
            inv_n = jnp.float32(1.0 / (D * F))
            w1 = w1_ref[0]
            s1 = 1.0 / jnp.clip(jnp.sum(jnp.abs(w1)) * inv_n, 1e-5)
            w1q_ref[...] = jnp.clip(jnp.round(w1 * s1), -1.0, 1.0) / s1
            w2 = w2_ref[0]
            s2 = 1.0 / jnp.clip(jnp.sum(jnp.abs(w2)) * inv_n, 1e-5)
            w2q_ref[...] = jnp.clip(jnp.round(w2 * s2), -1.0, 1.0) / s2

        x = xs_ref[...]
        sx = 127.0 / jnp.clip(jnp.max(jnp.abs(x), axis=1, keepdims=True), 1e-5)
        xq = jnp.clip(jnp.round(x * sx), -128.0, 127.0) / sx
        h = jnp.dot(xq, w1q_ref[...])
        g = 0.5 * h * (1.0 + jnp.tanh(_GELU_C * (h + 0.044715 * h * h * h)))
        sg = 127.0 / jnp.clip(jnp.max(jnp.abs(g), axis=1, keepdims=True), 1e-5)
        gq = jnp.clip(jnp.round(g * sg), -128.0, 127.0) / sg
        y = jnp.dot(gq, w2q_ref[...])
        ys_ref[...] = y


def _ffn(te, xs, w1, w2):
    grid_spec = pltpu.PrefetchScalarGridSpec(
        num_scalar_prefetch=1,
        grid=(NT,),
        in_specs=[
            pl.BlockSpec((TILE, D), lambda i, te: (i, 0)),
            pl.BlockSpec((1, D, F), lambda i, te: (te[i], 0, 0)),
            pl.BlockSpec((1, F, D), lambda i, te: (te[i], 0, 0)),
        ],
        out_specs=pl.BlockSpec((TILE, D), lambda i, te: (i, 0)),
        scratch_shapes=[
            pltpu.VMEM((D, F), jnp.float32),
            pltpu.VMEM((F, D), jnp.float32),
        ],
    )
    return pl.pallas_call(
        _ffn_body,
        grid_spec=grid_spec,
        out_shape=jax.ShapeDtypeStruct((NPAD, D), jnp.float32),
        compiler_params=pltpu.CompilerParams(
            dimension_semantics=("arbitrary",),
            vmem_limit_bytes=100 * 1024 * 1024,
        ),
    )(te, xs, w1, w2)


def kernel(x, router_w, w1, w2):
    sc_scatter, sc_gather = _sc_kernels()
    xf = x.reshape(N, D)
    logits = jnp.dot(xf, router_w)
    dest2d, te2d, aux = _router(logits)
    dest = dest2d.reshape(N)
    te = te2d.reshape(NT + 1)
    xs = sc_scatter(xf, dest)
    ys = _ffn(te, xs, w1, w2)
    out = sc_gather(ys, dest)
    return out.reshape(x.shape), aux.reshape(())
